# baseline (speedup 1.0000x reference)
import jax
import jax.numpy as jnp
from jax.experimental import pallas as pl
from jax.experimental.pallas import tpu as pltpu

_EMB = 2048
_NE = 64
_TOK = 16384
_COEFF = 0.001
_BT = 1024
_KSPLIT = 4
_KC = _EMB // _KSPLIT


def _router_kernel(x_ref, w_ref, b_ref, probs_ref, zpart_ref, acc_ref):
    j = pl.program_id(1)
    part = jax.lax.dot_general(
        x_ref[...], w_ref[...],
        dimension_numbers=(((1,), (1,)), ((), ())),
        preferred_element_type=jnp.float32,
    )

    @pl.when(j == 0)
    def _init():
        acc_ref[...] = part + b_ref[...]

    @pl.when(j > 0)
    def _acc():
        acc_ref[...] += part

    @pl.when(j == _KSPLIT - 1)
    def _fin():
        logits = acc_ref[...]
        m = jnp.max(logits, axis=-1, keepdims=True)
        e = jnp.exp(logits - m)
        s = jnp.sum(e, axis=-1, keepdims=True)
        probs_ref[...] = e / s
        zpart_ref[...] = jnp.sum(logits * logits).reshape(1, 1, 1)


def kernel(x, W, b):
    nblk = _TOK // _BT
    probs, zpart = pl.pallas_call(
        _router_kernel,
        grid=(nblk, _KSPLIT),
        in_specs=[
            pl.BlockSpec((_BT, _KC), lambda i, j: (i, j)),
            pl.BlockSpec((_NE, _KC), lambda i, j: (0, j)),
            pl.BlockSpec((1, _NE), lambda i, j: (0, 0)),
        ],
        out_specs=[
            pl.BlockSpec((_BT, _NE), lambda i, j: (i, 0)),
            pl.BlockSpec((1, 1, 1), lambda i, j: (i, 0, 0)),
        ],
        out_shape=[
            jax.ShapeDtypeStruct((_TOK, _NE), jnp.float32),
            jax.ShapeDtypeStruct((nblk, 1, 1), jnp.float32),
        ],
        scratch_shapes=[
            pltpu.VMEM((_BT, _NE), jnp.float32),
        ],
        compiler_params=pltpu.CompilerParams(
            dimension_semantics=("parallel", "arbitrary"),
        ),
    )(x, W, b.reshape(1, _NE))
    z_loss = jnp.sum(zpart) * (_COEFF / (_TOK * _NE))
    return (probs, z_loss)


# speedup vs baseline: 1.7528x; 1.7528x over previous
import jax
import jax.numpy as jnp
from jax.experimental import pallas as pl
from jax.experimental.pallas import tpu as pltpu

_EMB = 2048
_NE = 64
_TOK = 16384
_COEFF = 0.001
_BT = 1024


def _router_kernel(x_ref, w_ref, b_ref, probs_ref, z_ref, zacc_ref):
    i = pl.program_id(0)
    nblk = _TOK // _BT
    logits = jax.lax.dot_general(
        x_ref[...], w_ref[...],
        dimension_numbers=(((1,), (1,)), ((), ())),
        preferred_element_type=jnp.float32,
    ) + b_ref[...]
    m = jnp.max(logits, axis=-1, keepdims=True)
    e = jnp.exp(logits - m)
    s = jnp.sum(e, axis=-1, keepdims=True)
    probs_ref[...] = e / s
    part = jnp.sum(logits * logits).reshape(1, 1)

    @pl.when(i == 0)
    def _init():
        zacc_ref[...] = part

    @pl.when(i > 0)
    def _acc():
        zacc_ref[...] += part

    @pl.when(i == nblk - 1)
    def _fin():
        z_ref[...] = zacc_ref[...] * (_COEFF / (_TOK * _NE))


def kernel(x, W, b):
    nblk = _TOK // _BT
    probs, z = pl.pallas_call(
        _router_kernel,
        grid=(nblk,),
        in_specs=[
            pl.BlockSpec((_BT, _EMB), lambda i: (i, 0)),
            pl.BlockSpec((_NE, _EMB), lambda i: (0, 0)),
            pl.BlockSpec((1, _NE), lambda i: (0, 0)),
        ],
        out_specs=[
            pl.BlockSpec((_BT, _NE), lambda i: (i, 0)),
            pl.BlockSpec((1, 1), lambda i: (0, 0)),
        ],
        out_shape=[
            jax.ShapeDtypeStruct((_TOK, _NE), jnp.float32),
            jax.ShapeDtypeStruct((1, 1), jnp.float32),
        ],
        scratch_shapes=[
            pltpu.VMEM((1, 1), jnp.float32),
        ],
        compiler_params=pltpu.CompilerParams(
            dimension_semantics=("arbitrary",),
        ),
    )(x, W, b.reshape(1, _NE))
    return (probs, z.reshape(()))
